# baseline (speedup 1.0000x reference)
import functools

import jax
import jax.numpy as jnp
from jax import lax
from jax.experimental import pallas as pl
from jax.experimental.pallas import tpu as pltpu
from jax.experimental.pallas import tpu_sc as plsc

N, E, D, OUT = 10000, 320000, 128, 1

NC, NS, L = 2, 16, 16
NW = NC * NS
EDGES_PER_W = E // NW
BLK = 80
NBLK = EDGES_PER_W // BLK
GROUPS = BLK // L
DP = D + 1


def _project_body(h_ref, ws_ref, wd_ref, us_ref, ud_ref):
    h = h_ref[...]
    for w_ref, out_ref in ((ws_ref, us_ref), (wd_ref, ud_ref)):
        p = lax.dot_general(h, w_ref[...], (((1,), (1,)), ((), ())),
                            preferred_element_type=jnp.float32)
        inv = lax.rsqrt(jnp.sum(p * p, axis=1, keepdims=True))
        out_ref[...] = p * inv


def _project(h, w_src, w_dst):
    blk = 2000
    grid = N // blk
    return pl.pallas_call(
        _project_body,
        grid=(grid,),
        in_specs=[
            pl.BlockSpec((blk, D), lambda i: (i, 0)),
            pl.BlockSpec((D, D), lambda i: (0, 0)),
            pl.BlockSpec((D, D), lambda i: (0, 0)),
        ],
        out_specs=[
            pl.BlockSpec((blk, D), lambda i: (i, 0)),
            pl.BlockSpec((blk, D), lambda i: (i, 0)),
        ],
        out_shape=[
            jax.ShapeDtypeStruct((N, D), jnp.float32),
            jax.ShapeDtypeStruct((N, D), jnp.float32),
        ],
    )(h, w_src, w_dst)


def _edge_body(us_hbm, ud_hbm, src_hbm, dst_hbm, out_hbm,
               idx_s, idx_d, rows_s0, rows_d0, rows_s1, rows_d1, out_v,
               sem_s0, sem_d0, sem_s1, sem_d1):
    wid = lax.axis_index("s") * NC + lax.axis_index("c")
    lane = lax.iota(jnp.int32, L)
    base = wid * EDGES_PER_W

    pltpu.sync_copy(src_hbm.at[pl.ds(base, EDGES_PER_W)], idx_s)
    pltpu.sync_copy(dst_hbm.at[pl.ds(base, EDGES_PER_W)], idx_d)

    def issue(j, rows_s, rows_d, sem_s, sem_d):
        cs = pltpu.async_copy(us_hbm.at[idx_s.at[pl.ds(j * BLK, BLK)]],
                              rows_s, sem_s)
        cd = pltpu.async_copy(ud_hbm.at[idx_d.at[pl.ds(j * BLK, BLK)]],
                              rows_d, sem_d)
        return cs, cd

    def compute(j, rows_s, rows_d):
        @plsc.parallel_loop(0, GROUPS)
        def group_body(g):
            res = jnp.zeros((L,), jnp.float32)
            for i in range(L):
                e = g * L + i
                acc = rows_s[e, pl.ds(0, L)] * rows_d[e, pl.ds(0, L)]
                for k in range(1, D // L):
                    acc = acc + (rows_s[e, pl.ds(k * L, L)] *
                                 rows_d[e, pl.ds(k * L, L)])
                res = jnp.where(lane == i, jnp.sum(acc), res)
            out_v[pl.ds(j * BLK + g * L, L)] = res

    def wait(j, rows_s, rows_d, sem_s, sem_d):
        pltpu.make_async_copy(us_hbm.at[idx_s.at[pl.ds(j * BLK, BLK)]],
                              rows_s, sem_s).wait()
        pltpu.make_async_copy(ud_hbm.at[idx_d.at[pl.ds(j * BLK, BLK)]],
                              rows_d, sem_d).wait()

    issue(0, rows_s0, rows_d0, sem_s0, sem_d0)

    def pair_body(p, carry):
        j0 = 2 * p
        issue(j0 + 1, rows_s1, rows_d1, sem_s1, sem_d1)
        wait(j0, rows_s0, rows_d0, sem_s0, sem_d0)
        compute(j0, rows_s0, rows_d0)
        issue(j0 + 2, rows_s0, rows_d0, sem_s0, sem_d0)
        wait(j0 + 1, rows_s1, rows_d1, sem_s1, sem_d1)
        compute(j0 + 1, rows_s1, rows_d1)
        return carry

    lax.fori_loop(0, (NBLK - 1) // 2, pair_body, 0, unroll=False)
    wait(NBLK - 1, rows_s0, rows_d0, sem_s0, sem_d0)
    compute(NBLK - 1, rows_s0, rows_d0)

    pltpu.sync_copy(out_v, out_hbm.at[pl.ds(base, EDGES_PER_W)])


@functools.partial(jax.jit)
def _edge_scores(us, ud, src, dst):
    mesh = plsc.VectorSubcoreMesh(core_axis_name="c", subcore_axis_name="s",
                                  num_cores=NC, num_subcores=NS)
    return pl.kernel(
        _edge_body,
        out_type=jax.ShapeDtypeStruct((E,), jnp.float32),
        mesh=mesh,
        scratch_types=[
            pltpu.VMEM((EDGES_PER_W,), jnp.int32),
            pltpu.VMEM((EDGES_PER_W,), jnp.int32),
            pltpu.VMEM((BLK, D), jnp.float32),
            pltpu.VMEM((BLK, D), jnp.float32),
            pltpu.VMEM((BLK, D), jnp.float32),
            pltpu.VMEM((BLK, D), jnp.float32),
            pltpu.VMEM((EDGES_PER_W,), jnp.float32),
            pltpu.SemaphoreType.DMA,
            pltpu.SemaphoreType.DMA,
            pltpu.SemaphoreType.DMA,
            pltpu.SemaphoreType.DMA,
        ],
        compiler_params=pltpu.CompilerParams(needs_layout_passes=False,
                                             use_tc_tiling_on_sc=False),
    )(us, ud, src, dst)


def kernel(h, edge_index, W_src, W_dst):
    us, ud = _project(h, W_src, W_dst)
    src = edge_index[0]
    dst = edge_index[1]
    return _edge_scores(us, ud, src, dst)


# speedup vs baseline: 2.1061x; 2.1061x over previous
import functools

import jax
import jax.numpy as jnp
from jax import lax
from jax.experimental import pallas as pl
from jax.experimental.pallas import tpu as pltpu
from jax.experimental.pallas import tpu_sc as plsc

N, E, D, OUT = 10000, 320000, 128, 1

NC, NS, L = 2, 16, 16
NW = NC * NS
EDGES_PER_W = E // NW
BLK = 80
NBLK = EDGES_PER_W // BLK
GROUPS = BLK // L
WPN = D // 2


def _project_body(h_ref, ws_ref, wd_ref, us_ref, ud_ref):
    h = h_ref[...]
    for w_ref, out_ref in ((ws_ref, us_ref), (wd_ref, ud_ref)):
        p = lax.dot_general(h, w_ref[...], (((1,), (1,)), ((), ())),
                            preferred_element_type=jnp.float32)
        inv = lax.rsqrt(jnp.sum(p * p, axis=1, keepdims=True))
        out_ref[...] = p * inv


def _project(h, w_src, w_dst):
    blk = 2000
    grid = N // blk
    return pl.pallas_call(
        _project_body,
        grid=(grid,),
        in_specs=[
            pl.BlockSpec((blk, D), lambda i: (i, 0)),
            pl.BlockSpec((D, D), lambda i: (0, 0)),
            pl.BlockSpec((D, D), lambda i: (0, 0)),
        ],
        out_specs=[
            pl.BlockSpec((blk, D), lambda i: (i, 0)),
            pl.BlockSpec((blk, D), lambda i: (i, 0)),
        ],
        out_shape=[
            jax.ShapeDtypeStruct((N, D), jnp.float32),
            jax.ShapeDtypeStruct((N, D), jnp.float32),
        ],
    )(h, w_src, w_dst)


def _edge_body(us_hbm, ud_hbm, src_hbm, dst_hbm, out_hbm,
               idx_s, idx_d, rows_s0, rows_d0, rows_s1, rows_d1, out_v,
               sem_s0, sem_d0, sem_s1, sem_d1):
    wid = lax.axis_index("s") * NC + lax.axis_index("c")
    lane = lax.iota(jnp.int32, L)
    base = wid * EDGES_PER_W

    pltpu.sync_copy(src_hbm.at[pl.ds(base, EDGES_PER_W)], idx_s)
    pltpu.sync_copy(dst_hbm.at[pl.ds(base, EDGES_PER_W)], idx_d)

    def issue(j, rows_s, rows_d, sem_s, sem_d):
        cs = pltpu.async_copy(
            us_hbm.at[idx_s.at[pl.ds(j * BLK, BLK)]],
            rows_s, sem_s)
        cd = pltpu.async_copy(
            ud_hbm.at[idx_d.at[pl.ds(j * BLK, BLK)]],
            rows_d, sem_d)
        return cs, cd

    def compute(j, rows_s, rows_d):
        def group_body(g, c):
            res = jnp.zeros((L,), jnp.float32)
            for i in range(L):
                e = g * L + i
                acc = jnp.zeros((L,), jnp.float32)
                for k in range(WPN // L):
                    a = plsc.bitcast(rows_s[e, pl.ds(k * L, L)], jnp.bfloat16)
                    b = plsc.bitcast(rows_d[e, pl.ds(k * L, L)], jnp.bfloat16)
                    p0, p1 = plsc.unpack(a * b,
                                         format=plsc.PackFormat.INTERLEAVED)
                    acc = acc + p0 + p1
                res = jnp.where(lane == i, jnp.sum(acc), res)
            out_v[pl.ds(j * BLK + g * L, L)] = res
            return c

        lax.fori_loop(0, GROUPS, group_body, 0, unroll=False)

    def wait(j, rows_s, rows_d, sem_s, sem_d):
        pltpu.make_async_copy(
            us_hbm.at[idx_s.at[pl.ds(j * BLK, BLK)]],
            rows_s, sem_s).wait()
        pltpu.make_async_copy(
            ud_hbm.at[idx_d.at[pl.ds(j * BLK, BLK)]],
            rows_d, sem_d).wait()

    issue(0, rows_s0, rows_d0, sem_s0, sem_d0)

    def pair_body(p, carry):
        j0 = 2 * p
        issue(j0 + 1, rows_s1, rows_d1, sem_s1, sem_d1)
        wait(j0, rows_s0, rows_d0, sem_s0, sem_d0)
        compute(j0, rows_s0, rows_d0)
        issue(j0 + 2, rows_s0, rows_d0, sem_s0, sem_d0)
        wait(j0 + 1, rows_s1, rows_d1, sem_s1, sem_d1)
        compute(j0 + 1, rows_s1, rows_d1)
        return carry

    lax.fori_loop(0, (NBLK - 1) // 2, pair_body, 0, unroll=False)
    wait(NBLK - 1, rows_s0, rows_d0, sem_s0, sem_d0)
    compute(NBLK - 1, rows_s0, rows_d0)

    pltpu.sync_copy(out_v, out_hbm.at[pl.ds(base, EDGES_PER_W)])


@functools.partial(jax.jit)
def _edge_scores(us, ud, src, dst):
    mesh = plsc.VectorSubcoreMesh(core_axis_name="c", subcore_axis_name="s",
                                  num_cores=NC, num_subcores=NS)
    return pl.kernel(
        _edge_body,
        out_type=jax.ShapeDtypeStruct((E,), jnp.float32),
        mesh=mesh,
        scratch_types=[
            pltpu.VMEM((EDGES_PER_W,), jnp.int32),
            pltpu.VMEM((EDGES_PER_W,), jnp.int32),
            pltpu.VMEM((BLK, WPN), jnp.int32),
            pltpu.VMEM((BLK, WPN), jnp.int32),
            pltpu.VMEM((BLK, WPN), jnp.int32),
            pltpu.VMEM((BLK, WPN), jnp.int32),
            pltpu.VMEM((EDGES_PER_W,), jnp.float32),
            pltpu.SemaphoreType.DMA,
            pltpu.SemaphoreType.DMA,
            pltpu.SemaphoreType.DMA,
            pltpu.SemaphoreType.DMA,
        ],
        compiler_params=pltpu.CompilerParams(needs_layout_passes=False,
                                             use_tc_tiling_on_sc=False),
    )(us, ud, src, dst)


def _pack(u):
    ub = u.astype(jnp.bfloat16).reshape(N, WPN, 2)
    return lax.bitcast_convert_type(ub, jnp.int32)


def kernel(h, edge_index, W_src, W_dst):
    us, ud = _project(h, W_src, W_dst)
    src = edge_index[0]
    dst = edge_index[1]
    return _edge_scores(_pack(us), _pack(ud), src, dst)


# speedup vs baseline: 2.8194x; 1.3387x over previous
import functools

import jax
import jax.numpy as jnp
from jax import lax
from jax.experimental import pallas as pl
from jax.experimental.pallas import tpu as pltpu
from jax.experimental.pallas import tpu_sc as plsc

N, E, D, OUT = 10000, 320000, 128, 1

NC, NS, L = 2, 16, 16
NW = NC * NS
EDGES_PER_W = E // NW
BLK = 80
NBLK = EDGES_PER_W // BLK
GROUPS = BLK // L
WPN = D // 2


def _project_body(h_ref, ws_ref, wd_ref, us_ref, ud_ref):
    h = h_ref[...]
    for w_ref, out_ref in ((ws_ref, us_ref), (wd_ref, ud_ref)):
        p = lax.dot_general(h, w_ref[...], (((1,), (1,)), ((), ())),
                            preferred_element_type=jnp.float32)
        inv = lax.rsqrt(jnp.sum(p * p, axis=1, keepdims=True))
        u = (p * inv).astype(jnp.bfloat16)
        lo = lax.bitcast_convert_type(u[:, :WPN], jnp.uint16).astype(jnp.uint32)
        hi = lax.bitcast_convert_type(u[:, WPN:], jnp.uint16).astype(jnp.uint32)
        out_ref[...] = lax.bitcast_convert_type(lo | (hi << 16), jnp.int32)


def _project(h, w_src, w_dst):
    blk = 2000
    grid = N // blk
    return pl.pallas_call(
        _project_body,
        grid=(grid,),
        in_specs=[
            pl.BlockSpec((blk, D), lambda i: (i, 0)),
            pl.BlockSpec((D, D), lambda i: (0, 0)),
            pl.BlockSpec((D, D), lambda i: (0, 0)),
        ],
        out_specs=[
            pl.BlockSpec((blk, WPN), lambda i: (i, 0)),
            pl.BlockSpec((blk, WPN), lambda i: (i, 0)),
        ],
        out_shape=[
            jax.ShapeDtypeStruct((N, WPN), jnp.int32),
            jax.ShapeDtypeStruct((N, WPN), jnp.int32),
        ],
    )(h, w_src, w_dst)


def _edge_body(us_hbm, ud_hbm, src_hbm, dst_hbm, out_hbm,
               idx_s, idx_d, rows_s0, rows_d0, rows_s1, rows_d1, out_v,
               sem_s0, sem_d0, sem_s1, sem_d1):
    wid = lax.axis_index("s") * NC + lax.axis_index("c")
    lane = lax.iota(jnp.int32, L)
    base = wid * EDGES_PER_W

    pltpu.sync_copy(src_hbm.at[pl.ds(base, EDGES_PER_W)], idx_s)
    pltpu.sync_copy(dst_hbm.at[pl.ds(base, EDGES_PER_W)], idx_d)

    def issue(j, rows_s, rows_d, sem_s, sem_d):
        cs = pltpu.async_copy(
            us_hbm.at[idx_s.at[pl.ds(j * BLK, BLK)]],
            rows_s, sem_s)
        cd = pltpu.async_copy(
            ud_hbm.at[idx_d.at[pl.ds(j * BLK, BLK)]],
            rows_d, sem_d)
        return cs, cd

    def compute(j, rows_s, rows_d):
        def group_body(g, c):
            res = jnp.zeros((L,), jnp.float32)
            for i in range(L):
                e = g * L + i
                acc = jnp.zeros((L,), jnp.float32)
                for k in range(WPN // L):
                    a = plsc.bitcast(rows_s[e, pl.ds(k * L, L)], jnp.bfloat16)
                    b = plsc.bitcast(rows_d[e, pl.ds(k * L, L)], jnp.bfloat16)
                    p0, p1 = plsc.unpack(a * b,
                                         format=plsc.PackFormat.INTERLEAVED)
                    acc = acc + p0 + p1
                res = jnp.where(lane == i, jnp.sum(acc), res)
            out_v[pl.ds(j * BLK + g * L, L)] = res
            return c

        lax.fori_loop(0, GROUPS, group_body, 0, unroll=False)

    def wait(j, rows_s, rows_d, sem_s, sem_d):
        pltpu.make_async_copy(
            us_hbm.at[idx_s.at[pl.ds(j * BLK, BLK)]],
            rows_s, sem_s).wait()
        pltpu.make_async_copy(
            ud_hbm.at[idx_d.at[pl.ds(j * BLK, BLK)]],
            rows_d, sem_d).wait()

    issue(0, rows_s0, rows_d0, sem_s0, sem_d0)

    def pair_body(p, carry):
        j0 = 2 * p
        issue(j0 + 1, rows_s1, rows_d1, sem_s1, sem_d1)
        wait(j0, rows_s0, rows_d0, sem_s0, sem_d0)
        compute(j0, rows_s0, rows_d0)
        issue(j0 + 2, rows_s0, rows_d0, sem_s0, sem_d0)
        wait(j0 + 1, rows_s1, rows_d1, sem_s1, sem_d1)
        compute(j0 + 1, rows_s1, rows_d1)
        return carry

    lax.fori_loop(0, (NBLK - 1) // 2, pair_body, 0, unroll=False)
    wait(NBLK - 1, rows_s0, rows_d0, sem_s0, sem_d0)
    compute(NBLK - 1, rows_s0, rows_d0)

    pltpu.sync_copy(out_v, out_hbm.at[pl.ds(base, EDGES_PER_W)])


@functools.partial(jax.jit)
def _edge_scores(us, ud, src, dst):
    mesh = plsc.VectorSubcoreMesh(core_axis_name="c", subcore_axis_name="s",
                                  num_cores=NC, num_subcores=NS)
    return pl.kernel(
        _edge_body,
        out_type=jax.ShapeDtypeStruct((E,), jnp.float32),
        mesh=mesh,
        scratch_types=[
            pltpu.VMEM((EDGES_PER_W,), jnp.int32),
            pltpu.VMEM((EDGES_PER_W,), jnp.int32),
            pltpu.VMEM((BLK, WPN), jnp.int32),
            pltpu.VMEM((BLK, WPN), jnp.int32),
            pltpu.VMEM((BLK, WPN), jnp.int32),
            pltpu.VMEM((BLK, WPN), jnp.int32),
            pltpu.VMEM((EDGES_PER_W,), jnp.float32),
            pltpu.SemaphoreType.DMA,
            pltpu.SemaphoreType.DMA,
            pltpu.SemaphoreType.DMA,
            pltpu.SemaphoreType.DMA,
        ],
        compiler_params=pltpu.CompilerParams(needs_layout_passes=False,
                                             use_tc_tiling_on_sc=False),
    )(us, ud, src, dst)


def kernel(h, edge_index, W_src, W_dst):
    us, ud = _project(h, W_src, W_dst)
    src = edge_index[0]
    dst = edge_index[1]
    return _edge_scores(us, ud, src, dst)


# speedup vs baseline: 3.0418x; 1.0789x over previous
import functools

import jax
import jax.numpy as jnp
from jax import lax
from jax.experimental import pallas as pl
from jax.experimental.pallas import tpu as pltpu
from jax.experimental.pallas import tpu_sc as plsc

N, E, D, OUT = 10000, 320000, 128, 1

NC, NS, L = 2, 16, 16
NW = NC * NS
EDGES_PER_W = E // NW
BLK = 80
NBLK = EDGES_PER_W // BLK
GROUPS = BLK // L
WPN = D // 2


def _project_body(h_ref, ws_ref, wd_ref, us_ref, ud_ref):
    h = h_ref[...]
    for w_ref, out_ref in ((ws_ref, us_ref), (wd_ref, ud_ref)):
        p = lax.dot_general(h, w_ref[...], (((1,), (1,)), ((), ())),
                            preferred_element_type=jnp.float32)
        inv = lax.rsqrt(jnp.sum(p * p, axis=1, keepdims=True))
        u = (p * inv).astype(jnp.bfloat16)
        lo = lax.bitcast_convert_type(u[:, :WPN], jnp.uint16).astype(jnp.uint32)
        hi = lax.bitcast_convert_type(u[:, WPN:], jnp.uint16).astype(jnp.uint32)
        out_ref[...] = lax.bitcast_convert_type(lo | (hi << 16), jnp.int32)


def _project(h, w_src, w_dst):
    blk = 2000
    grid = N // blk
    return pl.pallas_call(
        _project_body,
        grid=(grid,),
        in_specs=[
            pl.BlockSpec((blk, D), lambda i: (i, 0)),
            pl.BlockSpec((D, D), lambda i: (0, 0)),
            pl.BlockSpec((D, D), lambda i: (0, 0)),
        ],
        out_specs=[
            pl.BlockSpec((blk, WPN), lambda i: (i, 0)),
            pl.BlockSpec((blk, WPN), lambda i: (i, 0)),
        ],
        out_shape=[
            jax.ShapeDtypeStruct((N, WPN), jnp.int32),
            jax.ShapeDtypeStruct((N, WPN), jnp.int32),
        ],
    )(h, w_src, w_dst)


def _edge_body(us_hbm, ud_hbm, ei_hbm, out_hbm,
               idx_s, idx_d, rows_s0, rows_d0, rows_s1, rows_d1, out_v,
               sem_s0, sem_d0, sem_s1, sem_d1):
    wid = lax.axis_index("s") * NC + lax.axis_index("c")
    lane = lax.iota(jnp.int32, L)
    base = wid * EDGES_PER_W

    pltpu.sync_copy(ei_hbm.at[0, pl.ds(base, EDGES_PER_W)], idx_s)
    pltpu.sync_copy(ei_hbm.at[1, pl.ds(base, EDGES_PER_W)], idx_d)

    def issue(j, rows_s, rows_d, sem_s, sem_d):
        cs = pltpu.async_copy(
            us_hbm.at[idx_s.at[pl.ds(j * BLK, BLK)]],
            rows_s, sem_s)
        cd = pltpu.async_copy(
            ud_hbm.at[idx_d.at[pl.ds(j * BLK, BLK)]],
            rows_d, sem_d)
        return cs, cd

    def compute(j, rows_s, rows_d):
        def group_body(g, c):
            res = jnp.zeros((L,), jnp.float32)
            for i in range(L):
                e = g * L + i
                acc = jnp.zeros((L,), jnp.float32)
                for k in range(WPN // L):
                    a = plsc.bitcast(rows_s[e, pl.ds(k * L, L)], jnp.bfloat16)
                    b = plsc.bitcast(rows_d[e, pl.ds(k * L, L)], jnp.bfloat16)
                    p0, p1 = plsc.unpack(a * b,
                                         format=plsc.PackFormat.INTERLEAVED)
                    acc = acc + p0 + p1
                res = jnp.where(lane == i, jnp.sum(acc), res)
            out_v[pl.ds(j * BLK + g * L, L)] = res
            return c

        lax.fori_loop(0, GROUPS, group_body, 0, unroll=False)

    def wait(j, rows_s, rows_d, sem_s, sem_d):
        pltpu.make_async_copy(
            us_hbm.at[idx_s.at[pl.ds(j * BLK, BLK)]],
            rows_s, sem_s).wait()
        pltpu.make_async_copy(
            ud_hbm.at[idx_d.at[pl.ds(j * BLK, BLK)]],
            rows_d, sem_d).wait()

    issue(0, rows_s0, rows_d0, sem_s0, sem_d0)

    def pair_body(p, carry):
        j0 = 2 * p
        issue(j0 + 1, rows_s1, rows_d1, sem_s1, sem_d1)
        wait(j0, rows_s0, rows_d0, sem_s0, sem_d0)
        compute(j0, rows_s0, rows_d0)
        issue(j0 + 2, rows_s0, rows_d0, sem_s0, sem_d0)
        wait(j0 + 1, rows_s1, rows_d1, sem_s1, sem_d1)
        compute(j0 + 1, rows_s1, rows_d1)
        return carry

    lax.fori_loop(0, (NBLK - 1) // 2, pair_body, 0, unroll=False)
    wait(NBLK - 1, rows_s0, rows_d0, sem_s0, sem_d0)
    compute(NBLK - 1, rows_s0, rows_d0)

    pltpu.sync_copy(out_v, out_hbm.at[pl.ds(base, EDGES_PER_W)])


@functools.partial(jax.jit)
def _edge_scores(us, ud, ei):
    mesh = plsc.VectorSubcoreMesh(core_axis_name="c", subcore_axis_name="s",
                                  num_cores=NC, num_subcores=NS)
    return pl.kernel(
        _edge_body,
        out_type=jax.ShapeDtypeStruct((E,), jnp.float32),
        mesh=mesh,
        scratch_types=[
            pltpu.VMEM((EDGES_PER_W,), jnp.int32),
            pltpu.VMEM((EDGES_PER_W,), jnp.int32),
            pltpu.VMEM((BLK, WPN), jnp.int32),
            pltpu.VMEM((BLK, WPN), jnp.int32),
            pltpu.VMEM((BLK, WPN), jnp.int32),
            pltpu.VMEM((BLK, WPN), jnp.int32),
            pltpu.VMEM((EDGES_PER_W,), jnp.float32),
            pltpu.SemaphoreType.DMA,
            pltpu.SemaphoreType.DMA,
            pltpu.SemaphoreType.DMA,
            pltpu.SemaphoreType.DMA,
        ],
        compiler_params=pltpu.CompilerParams(needs_layout_passes=False,
                                             use_tc_tiling_on_sc=False),
    )(us, ud, ei)


def kernel(h, edge_index, W_src, W_dst):
    us, ud = _project(h, W_src, W_dst)
    return _edge_scores(us, ud, edge_index)


# speedup vs baseline: 3.1267x; 1.0279x over previous
import functools

import jax
import jax.numpy as jnp
from jax import lax
from jax.experimental import pallas as pl
from jax.experimental.pallas import tpu as pltpu
from jax.experimental.pallas import tpu_sc as plsc

N, E, D, OUT = 10000, 320000, 128, 1

NC, NS, L = 2, 16, 16
NW = NC * NS
EDGES_PER_W = E // NW
BLK = 80
NBLK = EDGES_PER_W // BLK
GROUPS = BLK // L
WPN = D // 2


def _project_body(h_ref, ws_ref, wd_ref, us_ref, ud_ref):
    h = h_ref[...]
    for w_ref, out_ref in ((ws_ref, us_ref), (wd_ref, ud_ref)):
        p = lax.dot_general(h, w_ref[...], (((1,), (1,)), ((), ())),
                            preferred_element_type=jnp.float32)
        inv = lax.rsqrt(jnp.sum(p * p, axis=1, keepdims=True))
        u = (p * inv).astype(jnp.bfloat16)
        lo = lax.bitcast_convert_type(u[:, :WPN], jnp.uint16).astype(jnp.uint32)
        hi = lax.bitcast_convert_type(u[:, WPN:], jnp.uint16).astype(jnp.uint32)
        out_ref[...] = lax.bitcast_convert_type(lo | (hi << 16), jnp.int32)


def _project(h, w_src, w_dst):
    blk = 10000
    grid = N // blk
    return pl.pallas_call(
        _project_body,
        grid=(grid,),
        in_specs=[
            pl.BlockSpec((blk, D), lambda i: (i, 0)),
            pl.BlockSpec((D, D), lambda i: (0, 0)),
            pl.BlockSpec((D, D), lambda i: (0, 0)),
        ],
        out_specs=[
            pl.BlockSpec((blk, WPN), lambda i: (i, 0)),
            pl.BlockSpec((blk, WPN), lambda i: (i, 0)),
        ],
        out_shape=[
            jax.ShapeDtypeStruct((N, WPN), jnp.int32),
            jax.ShapeDtypeStruct((N, WPN), jnp.int32),
        ],
    )(h, w_src, w_dst)


def _edge_body(us_hbm, ud_hbm, ei_hbm, out_hbm,
               idx_s, idx_d, rows_s0, rows_d0, rows_s1, rows_d1, out_v,
               sem_s0, sem_d0, sem_s1, sem_d1):
    wid = lax.axis_index("s") * NC + lax.axis_index("c")
    lane = lax.iota(jnp.int32, L)
    base = wid * EDGES_PER_W

    pltpu.sync_copy(ei_hbm.at[0, pl.ds(base, EDGES_PER_W)], idx_s)
    pltpu.sync_copy(ei_hbm.at[1, pl.ds(base, EDGES_PER_W)], idx_d)

    def issue(j, rows_s, rows_d, sem_s, sem_d):
        cs = pltpu.async_copy(
            us_hbm.at[idx_s.at[pl.ds(j * BLK, BLK)]],
            rows_s, sem_s)
        cd = pltpu.async_copy(
            ud_hbm.at[idx_d.at[pl.ds(j * BLK, BLK)]],
            rows_d, sem_d)
        return cs, cd

    def compute(j, rows_s, rows_d):
        def group_body(g, c):
            res = jnp.zeros((L,), jnp.float32)
            for i in range(L):
                e = g * L + i
                acc = jnp.zeros((L,), jnp.float32)
                for k in range(WPN // L):
                    a = plsc.bitcast(rows_s[e, pl.ds(k * L, L)], jnp.bfloat16)
                    b = plsc.bitcast(rows_d[e, pl.ds(k * L, L)], jnp.bfloat16)
                    p0, p1 = plsc.unpack(a * b,
                                         format=plsc.PackFormat.INTERLEAVED)
                    acc = acc + p0 + p1
                res = jnp.where(lane == i, jnp.sum(acc), res)
            out_v[pl.ds(j * BLK + g * L, L)] = res
            return c

        lax.fori_loop(0, GROUPS, group_body, 0, unroll=False)

    def wait(j, rows_s, rows_d, sem_s, sem_d):
        pltpu.make_async_copy(
            us_hbm.at[idx_s.at[pl.ds(j * BLK, BLK)]],
            rows_s, sem_s).wait()
        pltpu.make_async_copy(
            ud_hbm.at[idx_d.at[pl.ds(j * BLK, BLK)]],
            rows_d, sem_d).wait()

    issue(0, rows_s0, rows_d0, sem_s0, sem_d0)

    def pair_body(p, carry):
        j0 = 2 * p
        issue(j0 + 1, rows_s1, rows_d1, sem_s1, sem_d1)
        wait(j0, rows_s0, rows_d0, sem_s0, sem_d0)
        compute(j0, rows_s0, rows_d0)
        issue(j0 + 2, rows_s0, rows_d0, sem_s0, sem_d0)
        wait(j0 + 1, rows_s1, rows_d1, sem_s1, sem_d1)
        compute(j0 + 1, rows_s1, rows_d1)
        return carry

    lax.fori_loop(0, (NBLK - 1) // 2, pair_body, 0, unroll=False)
    wait(NBLK - 1, rows_s0, rows_d0, sem_s0, sem_d0)
    compute(NBLK - 1, rows_s0, rows_d0)

    pltpu.sync_copy(out_v, out_hbm.at[pl.ds(base, EDGES_PER_W)])


@functools.partial(jax.jit)
def _edge_scores(us, ud, ei):
    mesh = plsc.VectorSubcoreMesh(core_axis_name="c", subcore_axis_name="s",
                                  num_cores=NC, num_subcores=NS)
    return pl.kernel(
        _edge_body,
        out_type=jax.ShapeDtypeStruct((E,), jnp.float32),
        mesh=mesh,
        scratch_types=[
            pltpu.VMEM((EDGES_PER_W,), jnp.int32),
            pltpu.VMEM((EDGES_PER_W,), jnp.int32),
            pltpu.VMEM((BLK, WPN), jnp.int32),
            pltpu.VMEM((BLK, WPN), jnp.int32),
            pltpu.VMEM((BLK, WPN), jnp.int32),
            pltpu.VMEM((BLK, WPN), jnp.int32),
            pltpu.VMEM((EDGES_PER_W,), jnp.float32),
            pltpu.SemaphoreType.DMA,
            pltpu.SemaphoreType.DMA,
            pltpu.SemaphoreType.DMA,
            pltpu.SemaphoreType.DMA,
        ],
        compiler_params=pltpu.CompilerParams(needs_layout_passes=False,
                                             use_tc_tiling_on_sc=False),
    )(us, ud, ei)


def kernel(h, edge_index, W_src, W_dst):
    us, ud = _project(h, W_src, W_dst)
    return _edge_scores(us, ud, edge_index)
